# Initial kernel scaffold; baseline (speedup 1.0000x reference)
import jax
import jax.numpy as jnp
from jax.experimental import pallas as pl


def kernel(inputs, table):
    raise NotImplementedError("write your pallas kernel here")


# speedup vs baseline: 1.1030x; 1.1030x over previous
import functools

import jax
import jax.numpy as jnp
from jax import lax
from jax.experimental import pallas as pl
from jax.experimental.pallas import tpu as pltpu
from jax.experimental.pallas import tpu_sc as plsc

_info = plsc.get_sparse_core_info()
_NC, _NS = _info.num_cores, _info.num_subcores
_NW = _NC * _NS


def _make_gather(n_rows: int, emb_dim: int, chunk: int):
    assert n_rows % (_NW * chunk) == 0
    n_per_w = n_rows // _NW
    n_chunks = n_per_w // chunk
    mesh = plsc.VectorSubcoreMesh(core_axis_name="c", subcore_axis_name="s")

    @functools.partial(
        pl.kernel,
        mesh=mesh,
        out_type=jax.ShapeDtypeStruct((n_rows, emb_dim), jnp.float32),
        scratch_types=[
            pltpu.VMEM((chunk,), jnp.int32),
            pltpu.VMEM((chunk, emb_dim), jnp.float32),
            pltpu.SemaphoreType.DMA,
        ],
        compiler_params=pltpu.CompilerParams(use_tc_tiling_on_sc=False),
    )
    def gather_kernel(idx_hbm, table_hbm, out_hbm, idx_v, rows_v, sem):
        wid = lax.axis_index("s") * _NC + lax.axis_index("c")
        base = wid * n_per_w

        def body(i, carry):
            off = base + i * chunk
            pltpu.sync_copy(idx_hbm.at[pl.ds(off, chunk)], idx_v)
            pltpu.async_copy(table_hbm.at[idx_v], rows_v, sem).wait()
            pltpu.sync_copy(rows_v, out_hbm.at[pl.ds(off, chunk)])
            return carry

        lax.fori_loop(0, n_chunks, body, 0)

    return gather_kernel


def kernel(inputs, table):
    batch, hist = inputs.shape
    n_vocab, emb_dim = table.shape
    idx = inputs.reshape(-1).astype(jnp.int32)
    n_rows = batch * hist
    flat = _make_gather(n_rows, emb_dim, chunk=1600)(idx, table)
    return flat.reshape(batch, hist, emb_dim)


# speedup vs baseline: 1.2249x; 1.1105x over previous
import functools

import jax
import jax.numpy as jnp
from jax import lax
from jax.experimental import pallas as pl
from jax.experimental.pallas import tpu as pltpu
from jax.experimental.pallas import tpu_sc as plsc

_info = plsc.get_sparse_core_info()
_NC, _NS = _info.num_cores, _info.num_subcores
_NW = _NC * _NS


def _make_gather(batch: int, hist: int, emb_dim: int, nb: int):
    rows_per_w = batch // _NW
    n_chunks = rows_per_w // nb
    assert batch % _NW == 0 and rows_per_w % nb == 0
    n_flat = nb * hist
    segs = list(range(0, hist - 15, 16))
    if hist % 16:
        segs.append(hist - 16)
    mesh = plsc.VectorSubcoreMesh(core_axis_name="c", subcore_axis_name="s")

    @functools.partial(
        pl.kernel,
        mesh=mesh,
        out_type=jax.ShapeDtypeStruct((batch * hist, emb_dim), jnp.float32),
        scratch_types=[
            pltpu.VMEM((nb, hist), jnp.int32),
            pltpu.VMEM((n_flat,), jnp.int32),
            pltpu.VMEM((n_flat, emb_dim), jnp.float32),
            pltpu.SemaphoreType.DMA,
        ],
        compiler_params=pltpu.CompilerParams(use_tc_tiling_on_sc=False),
    )
    def gather_kernel(idx_hbm, table_hbm, out_hbm, idx2_v, flat_v, rows_v, sem):
        wid = lax.axis_index("s") * _NC + lax.axis_index("c")
        base = wid * rows_per_w

        def body(i, carry):
            r0 = base + i * nb
            pltpu.sync_copy(idx_hbm.at[pl.ds(r0, nb), :], idx2_v)
            for r in range(nb):
                for c0 in segs:
                    flat_v[pl.ds(r * hist + c0, 16)] = idx2_v[r, pl.ds(c0, 16)]
            pltpu.async_copy(table_hbm.at[flat_v], rows_v, sem).wait()
            pltpu.sync_copy(rows_v, out_hbm.at[pl.ds(r0 * hist, n_flat)])
            return carry

        lax.fori_loop(0, n_chunks, body, 0)

    return gather_kernel


def _make_finisher(batch: int, hist: int, emb_dim: int, bb: int):
    assert batch % bb == 0
    grid = batch // bb

    def body(flat_ref, out_ref):
        out_ref[...] = flat_ref[...].reshape(bb, hist, emb_dim)

    return pl.pallas_call(
        body,
        grid=(grid,),
        in_specs=[pl.BlockSpec((bb * hist, emb_dim), lambda g: (g, 0))],
        out_specs=pl.BlockSpec((bb, hist, emb_dim), lambda g: (g, 0, 0)),
        out_shape=jax.ShapeDtypeStruct((batch, hist, emb_dim), jnp.float32),
    )


def kernel(inputs, table):
    batch, hist = inputs.shape
    n_vocab, emb_dim = table.shape
    flat = _make_gather(batch, hist, emb_dim, nb=32)(inputs, table)
    return _make_finisher(batch, hist, emb_dim, bb=64)(flat)
